# Initial kernel scaffold; baseline (speedup 1.0000x reference)
import jax
import jax.numpy as jnp
from jax.experimental import pallas as pl


def kernel(support_xf, support_y, query_xf, query_y):
    raise NotImplementedError("write your pallas kernel here")


# speedup vs baseline: 2.5751x; 2.5751x over previous
import functools

import jax
import jax.numpy as jnp
from jax.experimental import pallas as pl
from jax.experimental.pallas import tpu as pltpu

_N_WAY = 5
_K_SHOT = 5
_TEMP = 2.0


def _mn4_kernel(lab_ref, q_ref, s_ref, out_ref, *, n_shots, m, c, total):
    i = pl.program_id(0)

    s = s_ref[0]
    s = s.reshape(_N_WAY, n_shots, c, m).mean(axis=1)
    s_n = s / (jnp.sqrt(jnp.sum(s * s, axis=1, keepdims=True)) + 1e-8)

    q = q_ref[0]
    q_n = q / (jnp.sqrt(jnp.sum(q * q, axis=0, keepdims=True)) + 1e-8)

    iota_s = jax.lax.broadcasted_iota(jnp.int32, (m, m), 1)
    rmax_list = []
    bestv = None
    nstar = None
    astar = None
    for n in range(_N_WAY):
        s_cls = jax.lax.dot_general(
            q_n, s_n[n], (((0,), (0,)), ((), ())),
            preferred_element_type=jnp.float32)
        rmax_n = jnp.max(s_cls, axis=1, keepdims=True)
        an_n = jnp.min(jnp.where(s_cls == rmax_n, iota_s, m),
                       axis=1, keepdims=True)
        rmax_list.append(rmax_n)
        if n == 0:
            bestv = rmax_n
            nstar = jnp.zeros((m, 1), jnp.int32)
            astar = an_n
        else:
            upd = rmax_n > bestv
            bestv = jnp.where(upd, rmax_n, bestv)
            nstar = jnp.where(upd, n, nstar)
            astar = jnp.where(upd, an_n, astar)

    qn = nstar * m + astar

    qn_t = qn.T
    bestv_t = bestv.T
    row_i = jax.lax.broadcasted_iota(jnp.int32, (m, m), 0)
    col_i = iota_s
    same = qn == qn_t
    stronger = (bestv_t > bestv) | ((bestv_t == bestv) & (col_i < row_i))
    dom = jnp.max(jnp.where(same & stronger, 1, 0), axis=1, keepdims=True)
    mask = jnp.where((dom == 0) & (bestv > -1.0), _TEMP, 0.0)

    logits = [jnp.sum(r * mask) for r in rmax_list]
    pm = logits[0]
    for v in logits[1:]:
        pm = jnp.maximum(pm, v)
    lse = pm + jnp.log(sum(jnp.exp(v - pm) for v in logits))
    lab = lab_ref[i]
    picked = sum(jnp.where(lab == n, logits[n], 0.0) for n in range(_N_WAY))
    contrib = (lse - picked) * (1.0 / total)

    @pl.when(i == 0)
    def _init():
        out_ref[0, 0] = 0.0

    out_ref[0, 0] += contrib


def kernel(support_xf, support_y, query_xf, query_y):
    b, q_num, c, h, w = query_xf.shape
    m = h * w
    n_shots = support_xf.shape[1] // _N_WAY
    q_xf = query_xf.reshape(b * q_num, c, m)
    s_xf = support_xf.reshape(b, _N_WAY * n_shots, c, m)
    labels = query_y.reshape(b * q_num)
    total = b * q_num

    body = functools.partial(_mn4_kernel, n_shots=n_shots, m=m, c=c,
                             total=total)
    loss = pl.pallas_call(
        body,
        grid=(total,),
        in_specs=[
            pl.BlockSpec(memory_space=pltpu.SMEM),
            pl.BlockSpec((1, c, m), lambda i: (i, 0, 0)),
            pl.BlockSpec((1, _N_WAY * n_shots, c, m),
                         lambda i: (i // q_num, 0, 0, 0)),
        ],
        out_specs=pl.BlockSpec(memory_space=pltpu.SMEM),
        out_shape=jax.ShapeDtypeStruct((1, 1), jnp.float32),
    )(labels, q_xf, s_xf)
    return loss[0, 0]


# speedup vs baseline: 3.0359x; 1.1789x over previous
import functools

import jax
import jax.numpy as jnp
from jax.experimental import pallas as pl
from jax.experimental.pallas import tpu as pltpu

_N_WAY = 5
_K_SHOT = 5
_TEMP = 2.0


def _mn4_kernel(lab_ref, q_ref, s_ref, out_ref, sn_ref, *,
                n_shots, m, c, q_num, total):
    i = pl.program_id(0)

    @pl.when(i % q_num == 0)
    def _protos():
        s = s_ref[0]
        s = s.reshape(_N_WAY, n_shots, c, m).mean(axis=1)
        sn_ref[...] = s / (jnp.sqrt(jnp.sum(s * s, axis=1, keepdims=True))
                           + 1e-8)

    q = q_ref[0]
    q_n = q / (jnp.sqrt(jnp.sum(q * q, axis=0, keepdims=True)) + 1e-8)

    s_cls = []
    rmax = []
    for n in range(_N_WAY):
        t = jax.lax.dot_general(
            q_n, sn_ref[n], (((0,), (0,)), ((), ())),
            preferred_element_type=jnp.float32)
        s_cls.append(t)
        rmax.append(jnp.max(t, axis=1, keepdims=True))

    bestv = rmax[0]
    nstar = jnp.zeros((m, 1), jnp.float32)
    for n in range(1, _N_WAY):
        upd = rmax[n] > bestv
        bestv = jnp.where(upd, rmax[n], bestv)
        nstar = jnp.where(upd, float(n), nstar)

    s_best = s_cls[_N_WAY - 1]
    for n in range(_N_WAY - 2, -1, -1):
        s_best = jnp.where(nstar == float(n), s_cls[n], s_best)
    iota_s = jax.lax.broadcasted_iota(jnp.int32, (m, m), 1).astype(jnp.float32)
    astar = jnp.min(jnp.where(s_best == bestv, iota_s, 1e9),
                    axis=1, keepdims=True)
    qn = nstar * m + astar

    qn_t = qn.T
    bestv_t = bestv.T
    row_i = jax.lax.broadcasted_iota(jnp.int32, (m, m), 0).astype(jnp.float32)
    col_i = iota_s
    same = qn == qn_t
    stronger = (bestv_t > bestv) | ((bestv_t == bestv) & (col_i < row_i))
    dom = jnp.max(jnp.where(same & stronger, 1.0, 0.0), axis=1, keepdims=True)
    mask = jnp.where((dom == 0.0) & (bestv > -1.0), _TEMP, 0.0)

    logits = [jnp.sum(r * mask) for r in rmax]
    pm = logits[0]
    for v in logits[1:]:
        pm = jnp.maximum(pm, v)
    lse = pm + jnp.log(sum(jnp.exp(v - pm) for v in logits))
    lab = lab_ref[i]
    picked = sum(jnp.where(lab == n, logits[n], 0.0) for n in range(_N_WAY))
    contrib = (lse - picked) * (1.0 / total)

    @pl.when(i == 0)
    def _init():
        out_ref[0, 0] = 0.0

    out_ref[0, 0] += contrib


def kernel(support_xf, support_y, query_xf, query_y):
    b, q_num, c, h, w = query_xf.shape
    m = h * w
    n_shots = support_xf.shape[1] // _N_WAY
    q_xf = query_xf.reshape(b * q_num, c, m)
    s_xf = support_xf.reshape(b, _N_WAY * n_shots, c, m)
    labels = query_y.reshape(b * q_num)
    total = b * q_num

    body = functools.partial(_mn4_kernel, n_shots=n_shots, m=m, c=c,
                             q_num=q_num, total=total)
    loss = pl.pallas_call(
        body,
        grid=(total,),
        in_specs=[
            pl.BlockSpec(memory_space=pltpu.SMEM),
            pl.BlockSpec((1, c, m), lambda i: (i, 0, 0)),
            pl.BlockSpec((1, _N_WAY * n_shots, c, m),
                         lambda i: (i // q_num, 0, 0, 0)),
        ],
        out_specs=pl.BlockSpec(memory_space=pltpu.SMEM),
        out_shape=jax.ShapeDtypeStruct((1, 1), jnp.float32),
        scratch_shapes=[pltpu.VMEM((_N_WAY, c, m), jnp.float32)],
    )(labels, q_xf, s_xf)
    return loss[0, 0]


# speedup vs baseline: 3.0503x; 1.0048x over previous
import functools

import jax
import jax.numpy as jnp
from jax.experimental import pallas as pl
from jax.experimental.pallas import tpu as pltpu

_N_WAY = 5
_K_SHOT = 5
_TEMP = 2.0


def _mn4_kernel(*refs, b, n_shots, m, c, q_num, total):
    lab_ref, s_ref = refs[0], refs[1]
    q_refs = refs[2:2 + b]
    out_ref, sn_ref = refs[2 + b], refs[3 + b]
    i = pl.program_id(0)

    @pl.when(i == 0)
    def _protos():
        s = s_ref[...]
        s = s.reshape(b, _N_WAY, n_shots, c, m).mean(axis=2)
        sn_ref[...] = s / (jnp.sqrt(jnp.sum(s * s, axis=2, keepdims=True))
                           + 1e-8)
        out_ref[0, 0] = 0.0

    iota_s = jax.lax.broadcasted_iota(jnp.int32, (m, m), 1).astype(jnp.float32)
    row_i = jax.lax.broadcasted_iota(jnp.int32, (m, m), 0).astype(jnp.float32)
    iota_n = jax.lax.broadcasted_iota(jnp.int32, (1, _N_WAY), 1)

    contrib = 0.0
    for bb in range(b):
        q = q_refs[bb][0]
        q_n = q / (jnp.sqrt(jnp.sum(q * q, axis=0, keepdims=True)) + 1e-8)

        s_cls = []
        rmax = []
        for n in range(_N_WAY):
            t = jax.lax.dot_general(
                q_n, sn_ref[bb, n], (((0,), (0,)), ((), ())),
                preferred_element_type=jnp.float32)
            s_cls.append(t)
            rmax.append(jnp.max(t, axis=1, keepdims=True))

        bestv = rmax[0]
        nstar = jnp.zeros((m, 1), jnp.float32)
        for n in range(1, _N_WAY):
            upd = rmax[n] > bestv
            bestv = jnp.where(upd, rmax[n], bestv)
            nstar = jnp.where(upd, float(n), nstar)

        s_best = s_cls[_N_WAY - 1]
        for n in range(_N_WAY - 2, -1, -1):
            s_best = jnp.where(nstar == float(n), s_cls[n], s_best)
        astar = jnp.min(jnp.where(s_best == bestv, iota_s, 1e9),
                        axis=1, keepdims=True)
        qn = nstar * m + astar

        qn_t = qn.T
        bestv_t = bestv.T
        same = qn == qn_t
        stronger = (bestv_t > bestv) | ((bestv_t == bestv) & (iota_s < row_i))
        dom = jnp.max(jnp.where(same & stronger, 1.0, 0.0),
                      axis=1, keepdims=True)
        mask = jnp.where((dom == 0.0) & (bestv > -1.0), _TEMP, 0.0)

        rmax_cat = jnp.concatenate(rmax, axis=1)
        logits = jax.lax.dot_general(
            mask, rmax_cat, (((0,), (0,)), ((), ())),
            preferred_element_type=jnp.float32)
        pm = jnp.max(logits)
        lse = pm + jnp.log(jnp.sum(jnp.exp(logits - pm)))
        lab = lab_ref[bb * q_num + i]
        picked = jnp.sum(jnp.where(iota_n == lab, logits, 0.0))
        contrib = contrib + (lse - picked) * (1.0 / total)

    out_ref[0, 0] += contrib


def kernel(support_xf, support_y, query_xf, query_y):
    b, q_num, c, h, w = query_xf.shape
    m = h * w
    n_shots = support_xf.shape[1] // _N_WAY
    q_xf = query_xf.reshape(b * q_num, c, m)
    s_xf = support_xf.reshape(b, _N_WAY * n_shots, c, m)
    labels = query_y.reshape(b * q_num)
    total = b * q_num

    body = functools.partial(_mn4_kernel, b=b, n_shots=n_shots, m=m, c=c,
                             q_num=q_num, total=total)
    in_specs = [
        pl.BlockSpec(memory_space=pltpu.SMEM),
        pl.BlockSpec((b, _N_WAY * n_shots, c, m),
                     lambda i: (0, 0, 0, 0)),
    ]
    for bb in range(b):
        in_specs.append(
            pl.BlockSpec((1, c, m),
                         functools.partial(
                             lambda i, off: (off + i, 0, 0), off=bb * q_num)))
    loss = pl.pallas_call(
        body,
        grid=(q_num,),
        in_specs=in_specs,
        out_specs=pl.BlockSpec(memory_space=pltpu.SMEM),
        out_shape=jax.ShapeDtypeStruct((1, 1), jnp.float32),
        scratch_shapes=[pltpu.VMEM((b, _N_WAY, c, m), jnp.float32)],
    )(labels, s_xf, *([q_xf] * b))
    return loss[0, 0]


# speedup vs baseline: 3.8744x; 1.2701x over previous
import functools

import jax
import jax.numpy as jnp
from jax.experimental import pallas as pl
from jax.experimental.pallas import tpu as pltpu

_N_WAY = 5
_K_SHOT = 5
_TEMP = 2.0
_BIG = 1 << 20


def _mn4_kernel(*refs, b, n_shots, m, c, q_num, total):
    lab_ref, s_ref = refs[0], refs[1]
    q_refs = refs[2:2 + b]
    out_ref, sn_ref = refs[2 + b], refs[3 + b]
    i = pl.program_id(0)

    @pl.when(i == 0)
    def _protos():
        s = s_ref[...]
        s = s.reshape(b, _N_WAY, n_shots, c, m).mean(axis=2)
        sn_ref[...] = s / (jnp.sqrt(jnp.sum(s * s, axis=2, keepdims=True))
                           + 1e-8)
        out_ref[0, 0] = 0.0

    row_i = jax.lax.broadcasted_iota(jnp.int32, (m, m), 0)
    col_i = jax.lax.broadcasted_iota(jnp.int32, (m, m), 1)
    iota_nv = jax.lax.broadcasted_iota(jnp.int32, (_N_WAY, 1), 0)

    contrib = 0.0
    for bb in range(b):
        q = q_refs[bb][0]
        q_n = q / (jnp.sqrt(jnp.sum(q * q, axis=0, keepdims=True)) + 1e-8)

        s_cls = []
        rmax = []
        for n in range(_N_WAY):
            t = jax.lax.dot_general(
                sn_ref[bb, n], q_n, (((0,), (0,)), ((), ())),
                preferred_element_type=jnp.float32)
            s_cls.append(t)
            rmax.append(jnp.max(t, axis=0, keepdims=True))

        bestv = jnp.maximum(jnp.maximum(jnp.maximum(rmax[0], rmax[1]),
                                        jnp.maximum(rmax[2], rmax[3])),
                            rmax[4])

        codes = [jnp.min(jnp.where(s_cls[n] == bestv, row_i + n * m, _BIG),
                         axis=0, keepdims=True) for n in range(_N_WAY)]
        qn = jnp.minimum(jnp.minimum(jnp.minimum(codes[0], codes[1]),
                                     jnp.minimum(codes[2], codes[3])),
                         codes[4])

        qn_t = qn.T
        bestv_t = bestv.T
        same = qn_t == qn
        stronger = (bestv_t > bestv) | ((bestv_t == bestv) & (row_i < col_i))
        dom = jnp.any(same & stronger, axis=0, keepdims=True)
        mask = jnp.where((~dom) & (bestv > -1.0), _TEMP, 0.0)

        logits = jnp.concatenate(
            [jnp.sum(r * mask, axis=1, keepdims=True) for r in rmax],
            axis=0)
        pm = jnp.max(logits)
        lse = pm + jnp.log(jnp.sum(jnp.exp(logits - pm)))
        lab = lab_ref[bb * q_num + i]
        picked = jnp.sum(jnp.where(iota_nv == lab, logits, 0.0))
        contrib = contrib + (lse - picked) * (1.0 / total)

    out_ref[0, 0] += contrib


def kernel(support_xf, support_y, query_xf, query_y):
    b, q_num, c, h, w = query_xf.shape
    m = h * w
    n_shots = support_xf.shape[1] // _N_WAY
    q_xf = query_xf.reshape(b * q_num, c, m)
    s_xf = support_xf.reshape(b, _N_WAY * n_shots, c, m)
    labels = query_y.reshape(b * q_num)
    total = b * q_num

    body = functools.partial(_mn4_kernel, b=b, n_shots=n_shots, m=m, c=c,
                             q_num=q_num, total=total)
    in_specs = [
        pl.BlockSpec(memory_space=pltpu.SMEM),
        pl.BlockSpec((b, _N_WAY * n_shots, c, m),
                     lambda i: (0, 0, 0, 0)),
    ]
    for bb in range(b):
        in_specs.append(
            pl.BlockSpec((1, c, m),
                         functools.partial(
                             lambda i, off: (off + i, 0, 0), off=bb * q_num)))
    loss = pl.pallas_call(
        body,
        grid=(q_num,),
        in_specs=in_specs,
        out_specs=pl.BlockSpec(memory_space=pltpu.SMEM),
        out_shape=jax.ShapeDtypeStruct((1, 1), jnp.float32),
        scratch_shapes=[pltpu.VMEM((b, _N_WAY, c, m), jnp.float32)],
    )(labels, s_xf, *([q_xf] * b))
    return loss[0, 0]


# speedup vs baseline: 4.7434x; 1.2243x over previous
import functools

import jax
import jax.numpy as jnp
from jax.experimental import pallas as pl
from jax.experimental.pallas import tpu as pltpu

_N_WAY = 5
_K_SHOT = 5
_TEMP = 2.0
_BIG = 1 << 20
_QPB = 3


def _mn4_kernel(*refs, b, n_shots, m, c, q_num, total):
    lab_ref, s_ref = refs[0], refs[1]
    q_refs = refs[2:2 + b]
    out_ref, sn_ref = refs[2 + b], refs[3 + b]
    i = pl.program_id(0)

    @pl.when(i == 0)
    def _protos():
        s = s_ref[...]
        s = s.reshape(b, _N_WAY, n_shots, c, m).mean(axis=2)
        sn_ref[...] = s / (jnp.sqrt(jnp.sum(s * s, axis=2, keepdims=True))
                           + 1e-8)
        out_ref[0, 0] = 0.0

    row_i = jax.lax.broadcasted_iota(jnp.int32, (m, m), 0)
    col_i = jax.lax.broadcasted_iota(jnp.int32, (m, m), 1)
    iota_nv = jax.lax.broadcasted_iota(jnp.int32, (_N_WAY, 1), 0)

    contrib = 0.0
    for bb in range(b):
      for k in range(_QPB):
        q = q_refs[bb][k]
        q_n = q / (jnp.sqrt(jnp.sum(q * q, axis=0, keepdims=True)) + 1e-8)

        s_cls = []
        rmax = []
        for n in range(_N_WAY):
            t = jax.lax.dot_general(
                sn_ref[bb, n], q_n, (((0,), (0,)), ((), ())),
                preferred_element_type=jnp.float32)
            s_cls.append(t)
            rmax.append(jnp.max(t, axis=0, keepdims=True))

        bestv = jnp.maximum(jnp.maximum(jnp.maximum(rmax[0], rmax[1]),
                                        jnp.maximum(rmax[2], rmax[3])),
                            rmax[4])

        codes = [jnp.min(jnp.where(s_cls[n] == bestv, row_i + n * m, _BIG),
                         axis=0, keepdims=True) for n in range(_N_WAY)]
        qn = jnp.minimum(jnp.minimum(jnp.minimum(codes[0], codes[1]),
                                     jnp.minimum(codes[2], codes[3])),
                         codes[4])

        qn_t = qn.T
        bestv_t = bestv.T
        same = qn_t == qn
        stronger = (bestv_t > bestv) | ((bestv_t == bestv) & (row_i < col_i))
        dom = jnp.any(same & stronger, axis=0, keepdims=True)
        mask = jnp.where((~dom) & (bestv > -1.0), _TEMP, 0.0)

        logits = jnp.concatenate(
            [jnp.sum(r * mask, axis=1, keepdims=True) for r in rmax],
            axis=0)
        pm = jnp.max(logits)
        lse = pm + jnp.log(jnp.sum(jnp.exp(logits - pm)))
        lab = lab_ref[bb * q_num + i * _QPB + k]
        picked = jnp.sum(jnp.where(iota_nv == lab, logits, 0.0))
        contrib = contrib + (lse - picked) * (1.0 / total)

    out_ref[0, 0] += contrib


def kernel(support_xf, support_y, query_xf, query_y):
    b, q_num, c, h, w = query_xf.shape
    m = h * w
    n_shots = support_xf.shape[1] // _N_WAY
    q_xf = query_xf.reshape(b * q_num, c, m)
    s_xf = support_xf.reshape(b, _N_WAY * n_shots, c, m)
    labels = query_y.reshape(b * q_num)
    total = b * q_num

    body = functools.partial(_mn4_kernel, b=b, n_shots=n_shots, m=m, c=c,
                             q_num=q_num, total=total)
    in_specs = [
        pl.BlockSpec(memory_space=pltpu.SMEM),
        pl.BlockSpec((b, _N_WAY * n_shots, c, m),
                     lambda i: (0, 0, 0, 0)),
    ]
    for bb in range(b):
        in_specs.append(
            pl.BlockSpec((_QPB, c, m),
                         functools.partial(
                             lambda i, off: (off + i, 0, 0),
                             off=bb * q_num // _QPB)))
    loss = pl.pallas_call(
        body,
        grid=(q_num // _QPB,),
        in_specs=in_specs,
        out_specs=pl.BlockSpec(memory_space=pltpu.SMEM),
        out_shape=jax.ShapeDtypeStruct((1, 1), jnp.float32),
        scratch_shapes=[pltpu.VMEM((b, _N_WAY, c, m), jnp.float32)],
    )(labels, s_xf, *([q_xf] * b))
    return loss[0, 0]
